# Initial kernel scaffold; baseline (speedup 1.0000x reference)
import jax
import jax.numpy as jnp
from jax.experimental import pallas as pl


def kernel(notes, edge_index, edge_weight, w, b, garment_size):
    raise NotImplementedError("write your pallas kernel here")


# speedup vs baseline: 4.2922x; 4.2922x over previous
import functools

import jax
import jax.numpy as jnp
from jax import lax
from jax.experimental import pallas as pl
from jax.experimental.pallas import tpu as pltpu, tpu_sc as plsc

N = 10000
E = 320000
D = 128
GSZ = 8000
TAIL = N - GSZ

NC, NS = 2, 16
NW = NC * NS
EPW = E // NW
K = 80
CH = EPW // K
RPT = 624
ZR = 208
REM = N - NS * RPT

MB = 400


def _mm_body(notes_ref, w_ref, o_ref):
    o_ref[...] = jnp.dot(notes_ref[...], w_ref[...],
                         preferred_element_type=jnp.float32)


def _matmul(notes, w):
    return pl.pallas_call(
        _mm_body,
        grid=(N // MB,),
        in_specs=[
            pl.BlockSpec((MB, D), lambda i: (i, 0)),
            pl.BlockSpec((D, D), lambda i: (0, 0)),
        ],
        out_specs=pl.BlockSpec((MB, D), lambda i: (i, 0)),
        out_shape=jax.ShapeDtypeStruct((N, D), jnp.float32),
    )(notes, w)


def _sc_scatter_fn():
    mesh = plsc.VectorSubcoreMesh(
        core_axis_name="c", subcore_axis_name="s",
        num_cores=NC, num_subcores=NS)

    @functools.partial(
        pl.kernel,
        out_type=jax.ShapeDtypeStruct((NC, N, D), jnp.float32),
        mesh=mesh,
        scratch_types=[
            pltpu.VMEM((K,), jnp.int32),
            pltpu.VMEM((K,), jnp.int32),
            pltpu.VMEM((K,), jnp.float32),
            pltpu.VMEM((K, D), jnp.float32),
            pltpu.VMEM((ZR, D), jnp.float32),
            pltpu.VMEM_SHARED((N, D), jnp.float32),
            pltpu.SemaphoreType.DMA,
        ],
    )
    def sc_scatter(x0_hbm, src_hbm, dst_hbm, ew_hbm, part_hbm,
                   src_v, dst_v, ew_v, rows_v, zbuf, acc, sem):
        c = lax.axis_index("c")
        s = lax.axis_index("s")
        wid = c * NS + s

        @pl.loop(0, ZR)
        def _(r):
            for g in range(D // 16):
                zbuf[r, pl.ds(g * 16, 16)] = jnp.zeros((16,), jnp.float32)

        @pl.loop(0, RPT // ZR)
        def _(j):
            pltpu.sync_copy(zbuf, acc.at[pl.ds(s * RPT + j * ZR, ZR)])

        @pl.when(s == NS - 1)
        def _():
            pltpu.sync_copy(zbuf.at[pl.ds(0, REM)],
                            acc.at[pl.ds(NS * RPT, REM)])

        plsc.subcore_barrier()

        base = wid * EPW

        @pl.loop(0, CH)
        def _(i):
            off = base + i * K
            pltpu.sync_copy(src_hbm.at[pl.ds(off, K)], src_v)
            pltpu.sync_copy(dst_hbm.at[pl.ds(off, K)], dst_v)
            pltpu.sync_copy(ew_hbm.at[pl.ds(off, K)], ew_v)
            pltpu.async_copy(x0_hbm.at[src_v], rows_v, sem).wait()

            @pl.loop(0, K // 16)
            def _(eb):
                wchunk = ew_v[pl.ds(eb * 16, 16)]
                for l in range(16):
                    wv = jnp.full((16,), 0.0, jnp.float32) + wchunk[l]
                    e = eb * 16 + l
                    for g in range(D // 16):
                        sl = pl.ds(g * 16, 16)
                        rows_v[e, sl] = rows_v[e, sl] * wv

            pltpu.sync_copy(rows_v, acc.at[dst_v], add=True)

        plsc.subcore_barrier()

        @pl.loop(0, RPT // ZR)
        def _(j):
            r0 = s * RPT + j * ZR
            pltpu.sync_copy(acc.at[pl.ds(r0, ZR)],
                            part_hbm.at[c, pl.ds(r0, ZR)])

        @pl.when(s == NS - 1)
        def _():
            pltpu.sync_copy(acc.at[pl.ds(NS * RPT, REM)],
                            part_hbm.at[c, pl.ds(NS * RPT, REM)])

    return sc_scatter


_sc_scatter = _sc_scatter_fn()


def _merge_body(parts_ref, x0_ref, b_ref, o_ref):
    i = pl.program_id(0)

    @pl.when(i < N // MB)
    def _():
        o_ref[...] = jnp.maximum(
            parts_ref[0] + parts_ref[1] + b_ref[...], 0.0)

    @pl.when(i >= N // MB)
    def _():
        o_ref[...] = x0_ref[...]


def _merge(parts, x0, b):
    nblk = (N + TAIL) // MB
    return pl.pallas_call(
        _merge_body,
        grid=(nblk,),
        in_specs=[
            pl.BlockSpec((NC, MB, D),
                         lambda i: (0, jnp.minimum(i, N // MB - 1), 0)),
            pl.BlockSpec((MB, D),
                         lambda i: (GSZ // MB + jnp.maximum(i - N // MB, 0), 0)),
            pl.BlockSpec((1, D), lambda i: (0, 0)),
        ],
        out_specs=pl.BlockSpec((MB, D), lambda i: (i, 0)),
        out_shape=jax.ShapeDtypeStruct((N + TAIL, D), jnp.float32),
    )(parts, x0, b)


def kernel(notes, edge_index, edge_weight, w, b, garment_size):
    del garment_size
    dst = edge_index[0].astype(jnp.int32)
    src = edge_index[1].astype(jnp.int32)
    x0 = _matmul(notes, w)
    parts = _sc_scatter(x0, src, dst, edge_weight)
    return _merge(parts, x0, b.reshape(1, D))


# speedup vs baseline: 8.8226x; 2.0555x over previous
import functools

import jax
import jax.numpy as jnp
from jax import lax
from jax.experimental import pallas as pl
from jax.experimental.pallas import tpu as pltpu, tpu_sc as plsc

N = 10000
E = 320000
D = 128
GSZ = 8000
TAIL = N - GSZ

NC, NS = 2, 16
NW = NC * NS
EPW = E // NW
K = 80
CH = EPW // K
RPT = 624
ZR = 48
REM = N - NS * RPT

MB = 400


def _mm_body(notes_ref, w_ref, o_ref):
    o_ref[...] = jnp.dot(notes_ref[...], w_ref[...],
                         preferred_element_type=jnp.float32)


def _matmul(notes, w):
    return pl.pallas_call(
        _mm_body,
        grid=(N // MB,),
        in_specs=[
            pl.BlockSpec((MB, D), lambda i: (i, 0)),
            pl.BlockSpec((D, D), lambda i: (0, 0)),
        ],
        out_specs=pl.BlockSpec((MB, D), lambda i: (i, 0)),
        out_shape=jax.ShapeDtypeStruct((N, D), jnp.float32),
    )(notes, w)


def _sc_scatter_fn():
    mesh = plsc.VectorSubcoreMesh(
        core_axis_name="c", subcore_axis_name="s",
        num_cores=NC, num_subcores=NS)

    @functools.partial(
        pl.kernel,
        out_type=jax.ShapeDtypeStruct((NC, N, D), jnp.float32),
        mesh=mesh,
        scratch_types=[
            pltpu.VMEM((CH, K), jnp.int32),
            pltpu.VMEM((K,), jnp.int32),
            pltpu.VMEM((K,), jnp.int32),
            pltpu.VMEM((K,), jnp.int32),
            pltpu.VMEM((K,), jnp.int32),
            pltpu.VMEM((1, K), jnp.float32),
            pltpu.VMEM((1, K), jnp.float32),
            pltpu.VMEM((K, D), jnp.float32),
            pltpu.VMEM((K, D), jnp.float32),
            pltpu.VMEM((ZR, D), jnp.float32),
            pltpu.VMEM_SHARED((N, D), jnp.float32),
            pltpu.SemaphoreType.DMA,
            pltpu.SemaphoreType.DMA,
        ],
    )
    def sc_scatter(x0_hbm, packed_hbm, ew_hbm, part_hbm,
                   packed_v, src0, src1, dst0, dst1, ew0, ew1,
                   rows0, rows1, zbuf, acc, sem0, sem1):
        c = lax.axis_index("c")
        s = lax.axis_index("s")
        wid = c * NS + s
        rows = (rows0, rows1)
        srcb = (src0, src1)
        dstb = (dst0, dst1)
        ewb = (ew0, ew1)
        sems = (sem0, sem1)

        pltpu.sync_copy(packed_hbm.at[wid], packed_v)

        @pl.loop(0, ZR)
        def _(r):
            for g in range(D // 16):
                zbuf[r, pl.ds(g * 16, 16)] = jnp.zeros((16,), jnp.float32)

        @pl.loop(0, RPT // ZR)
        def _(j):
            pltpu.sync_copy(zbuf, acc.at[pl.ds(s * RPT + j * ZR, ZR)])

        @pl.when(s == NS - 1)
        def _():
            pltpu.sync_copy(zbuf.at[pl.ds(0, REM)],
                            acc.at[pl.ds(NS * RPT, REM)])

        def unpack(i, b):
            for v in range(K // 16):
                sl = pl.ds(v * 16, 16)
                p = packed_v[i, sl]
                srcb[b][sl] = jnp.bitwise_and(p, 0xFFFF)
                dstb[b][sl] = jnp.right_shift(p, 16)

        def issue(i, b):
            pltpu.async_copy(x0_hbm.at[srcb[b]], rows[b], sems[b])
            pltpu.async_copy(ew_hbm.at[wid, pl.ds(i, 1)], ewb[b], sems[b])

        def wait(i, b):
            pltpu.make_async_copy(
                x0_hbm.at[srcb[b]], rows[b], sems[b]).wait()
            pltpu.make_async_copy(
                ew_hbm.at[wid, pl.ds(i, 1)], ewb[b], sems[b]).wait()

        def weight_rows(b):
            @pl.loop(0, K // 16)
            def _(eb):
                wchunk = ewb[b][0, pl.ds(eb * 16, 16)]
                for l in range(16):
                    wv = jnp.full((16,), 0.0, jnp.float32) + wchunk[l]
                    e = eb * 16 + l
                    for g in range(D // 16):
                        sl = pl.ds(g * 16, 16)
                        rows[b][e, sl] = rows[b][e, sl] * wv

        unpack(0, 0)
        issue(0, 0)

        plsc.subcore_barrier()

        @pl.loop(0, (CH - 1) // 2)
        def _(j):
            for b in range(2):
                i = 2 * j + b
                wait(i, b)
                unpack(i + 1, 1 - b)
                issue(i + 1, 1 - b)
                weight_rows(b)
                pltpu.sync_copy(rows[b], acc.at[dstb[b]], add=True)

        wait(CH - 1, 0)
        weight_rows(0)
        pltpu.sync_copy(rows0, acc.at[dst0], add=True)

        plsc.subcore_barrier()

        @pl.loop(0, RPT // ZR)
        def _(j):
            r0 = s * RPT + j * ZR
            pltpu.sync_copy(acc.at[pl.ds(r0, ZR)],
                            part_hbm.at[c, pl.ds(r0, ZR)])

        @pl.when(s == NS - 1)
        def _():
            pltpu.sync_copy(acc.at[pl.ds(NS * RPT, REM)],
                            part_hbm.at[c, pl.ds(NS * RPT, REM)])

    return sc_scatter


_sc_scatter = _sc_scatter_fn()


def _merge_body(parts_ref, x0_ref, b_ref, o_ref):
    i = pl.program_id(0)

    @pl.when(i < N // MB)
    def _():
        o_ref[...] = jnp.maximum(
            parts_ref[0] + parts_ref[1] + b_ref[...], 0.0)

    @pl.when(i >= N // MB)
    def _():
        o_ref[...] = x0_ref[...]


def _merge(parts, x0, b):
    nblk = (N + TAIL) // MB
    return pl.pallas_call(
        _merge_body,
        grid=(nblk,),
        in_specs=[
            pl.BlockSpec((NC, MB, D),
                         lambda i: (0, jnp.minimum(i, N // MB - 1), 0)),
            pl.BlockSpec((MB, D),
                         lambda i: (GSZ // MB + jnp.maximum(i - N // MB, 0), 0)),
            pl.BlockSpec((1, D), lambda i: (0, 0)),
        ],
        out_specs=pl.BlockSpec((MB, D), lambda i: (i, 0)),
        out_shape=jax.ShapeDtypeStruct((N + TAIL, D), jnp.float32),
    )(parts, x0, b)


def kernel(notes, edge_index, edge_weight, w, b, garment_size):
    del garment_size
    dst = edge_index[0].astype(jnp.int32)
    src = edge_index[1].astype(jnp.int32)
    packed = jnp.bitwise_or(jnp.left_shift(dst, 16), src).reshape(NW, CH, K)
    ew = edge_weight.reshape(NW, CH, K)
    x0 = _matmul(notes, w)
    parts = _sc_scatter(x0, packed, ew)
    return _merge(parts, x0, b.reshape(1, D))


# speedup vs baseline: 9.6619x; 1.0951x over previous
import functools

import jax
import jax.numpy as jnp
from jax import lax
from jax.experimental import pallas as pl
from jax.experimental.pallas import tpu as pltpu, tpu_sc as plsc

N = 10000
E = 320000
D = 128
GSZ = 8000
TAIL = N - GSZ

NC, NS = 2, 16
NW = NC * NS
EPW = E // NW
K = 80
CH = EPW // K
RPT = 624
ZR = 24
REM = N - NS * RPT

MB = 2000
ECOL = E // 500


def _mm_body(notes_ref, w_ref, ei_ref, o_ref, p_ref):
    o_ref[...] = jnp.dot(notes_ref[...], w_ref[...],
                         preferred_element_type=jnp.float32)
    p_ref[...] = jnp.bitwise_or(
        jnp.left_shift(ei_ref[0], 16), ei_ref[1])


def _matmul_pack(notes, w, ei):
    return pl.pallas_call(
        _mm_body,
        grid=(N // MB,),
        in_specs=[
            pl.BlockSpec((MB, D), lambda i: (i, 0)),
            pl.BlockSpec((D, D), lambda i: (0, 0)),
            pl.BlockSpec((2, 500, 128), lambda i: (0, 0, i)),
        ],
        out_specs=[
            pl.BlockSpec((MB, D), lambda i: (i, 0)),
            pl.BlockSpec((500, 128), lambda i: (0, i)),
        ],
        out_shape=[
            jax.ShapeDtypeStruct((N, D), jnp.float32),
            jax.ShapeDtypeStruct((500, ECOL), jnp.int32),
        ],
    )(notes, w, ei)


def _sc_scatter_fn():
    mesh = plsc.VectorSubcoreMesh(
        core_axis_name="c", subcore_axis_name="s",
        num_cores=NC, num_subcores=NS)

    @functools.partial(
        pl.kernel,
        out_type=jax.ShapeDtypeStruct((NC, N, D), jnp.float32),
        mesh=mesh,
        scratch_types=[
            pltpu.VMEM((CH, K), jnp.int32),
            pltpu.VMEM((K,), jnp.int32),
            pltpu.VMEM((K,), jnp.int32),
            pltpu.VMEM((K,), jnp.int32),
            pltpu.VMEM((K,), jnp.int32),
            pltpu.VMEM((1, K), jnp.float32),
            pltpu.VMEM((1, K), jnp.float32),
            pltpu.VMEM((K, D), jnp.float32),
            pltpu.VMEM((K, D), jnp.float32),
            pltpu.VMEM((ZR, D), jnp.float32),
            pltpu.VMEM_SHARED((N, D), jnp.float32),
            pltpu.SemaphoreType.DMA,
            pltpu.SemaphoreType.DMA,
            pltpu.SemaphoreType.DMA,
            pltpu.SemaphoreType.DMA,
        ],
    )
    def sc_scatter(x0_hbm, packed_hbm, ew_hbm, part_hbm,
                   packed_v, src0, src1, dst0, dst1, ew0, ew1,
                   rows0, rows1, zbuf, acc,
                   gsem0, gsem1, ssem0, ssem1):
        c = lax.axis_index("c")
        s = lax.axis_index("s")
        wid = c * NS + s
        rows = (rows0, rows1)
        srcb = (src0, src1)
        dstb = (dst0, dst1)
        ewb = (ew0, ew1)
        gsems = (gsem0, gsem1)
        ssems = (ssem0, ssem1)

        pltpu.sync_copy(packed_hbm.at[wid], packed_v)

        @pl.loop(0, ZR)
        def _(r):
            for g in range(D // 16):
                zbuf[r, pl.ds(g * 16, 16)] = jnp.zeros((16,), jnp.float32)

        @pl.loop(0, RPT // ZR)
        def _(j):
            pltpu.sync_copy(zbuf, acc.at[pl.ds(s * RPT + j * ZR, ZR)])

        @pl.when(s == NS - 1)
        def _():
            pltpu.sync_copy(zbuf.at[pl.ds(0, REM)],
                            acc.at[pl.ds(NS * RPT, REM)])

        def unpack(i, b):
            for v in range(K // 16):
                sl = pl.ds(v * 16, 16)
                p = packed_v[i, sl]
                srcb[b][sl] = jnp.bitwise_and(p, 0xFFFF)
                dstb[b][sl] = jnp.right_shift(p, 16)

        def issue_gather(i, b):
            pltpu.async_copy(x0_hbm.at[srcb[b]], rows[b], gsems[b])
            pltpu.async_copy(ew_hbm.at[wid, pl.ds(i, 1)], ewb[b], gsems[b])

        def wait_gather(i, b):
            pltpu.make_async_copy(
                x0_hbm.at[srcb[b]], rows[b], gsems[b]).wait()
            pltpu.make_async_copy(
                ew_hbm.at[wid, pl.ds(i, 1)], ewb[b], gsems[b]).wait()

        def issue_scatter(b):
            pltpu.async_copy(rows[b], acc.at[dstb[b]], ssems[b], add=True)

        def wait_scatter(b):
            pltpu.make_async_copy(
                rows[b], acc.at[dstb[b]], ssems[b]).wait()

        def weight_rows(b):
            @pl.loop(0, K // 16)
            def _(eb):
                wchunk = ewb[b][0, pl.ds(eb * 16, 16)]
                for l in range(16):
                    wv = jnp.full((16,), 0.0, jnp.float32) + wchunk[l]
                    e = eb * 16 + l
                    for g in range(D // 16):
                        sl = pl.ds(g * 16, 16)
                        rows[b][e, sl] = rows[b][e, sl] * wv

        unpack(0, 0)
        issue_gather(0, 0)
        plsc.subcore_barrier()

        wait_gather(0, 0)
        unpack(1, 1)
        issue_gather(1, 1)
        weight_rows(0)
        issue_scatter(0)

        @pl.loop(0, (CH - 1) // 2)
        def _(j):
            for bi in range(2):
                i = 1 + 2 * j + bi
                b = (1 + bi) % 2
                wait_gather(i, b)
                wait_scatter(1 - b)
                i1 = jnp.minimum(i + 1, CH - 1)
                unpack(i1, 1 - b)
                issue_gather(i1, 1 - b)
                weight_rows(b)
                issue_scatter(b)

        wait_scatter(0)
        wait_gather(CH - 1, 1)

        plsc.subcore_barrier()

        @pl.loop(0, RPT // ZR)
        def _(j):
            r0 = s * RPT + j * ZR
            pltpu.sync_copy(acc.at[pl.ds(r0, ZR)],
                            part_hbm.at[c, pl.ds(r0, ZR)])

        @pl.when(s == NS - 1)
        def _():
            pltpu.sync_copy(acc.at[pl.ds(NS * RPT, REM)],
                            part_hbm.at[c, pl.ds(NS * RPT, REM)])

    return sc_scatter


_sc_scatter = _sc_scatter_fn()


def _merge_body(parts_ref, notes_ref, w_ref, b_ref, o_ref):
    i = pl.program_id(0)

    @pl.when(i < N // MB)
    def _():
        o_ref[...] = jnp.maximum(
            parts_ref[0] + parts_ref[1] + b_ref[...], 0.0)

    @pl.when(i >= N // MB)
    def _():
        o_ref[...] = jnp.dot(notes_ref[...], w_ref[...],
                             preferred_element_type=jnp.float32)


def _merge(parts, notes, w, b):
    nblk = (N + TAIL) // MB
    return pl.pallas_call(
        _merge_body,
        grid=(nblk,),
        in_specs=[
            pl.BlockSpec((NC, MB, D),
                         lambda i: (0, jnp.minimum(i, N // MB - 1), 0)),
            pl.BlockSpec((MB, D),
                         lambda i: (jnp.where(i >= N // MB, GSZ // MB, 0), 0)),
            pl.BlockSpec((D, D), lambda i: (0, 0)),
            pl.BlockSpec((1, D), lambda i: (0, 0)),
        ],
        out_specs=pl.BlockSpec((MB, D), lambda i: (i, 0)),
        out_shape=jax.ShapeDtypeStruct((N + TAIL, D), jnp.float32),
    )(parts, notes, w, b)


def kernel(notes, edge_index, edge_weight, w, b, garment_size):
    del garment_size
    ei = edge_index.astype(jnp.int32).reshape(2, 500, ECOL)
    ew = edge_weight.reshape(NW, CH, K)
    x0, packed = _matmul_pack(notes, w, ei)
    parts = _sc_scatter(x0, packed.reshape(NW, CH, K), ew)
    return _merge(parts, notes, w, b.reshape(1, D))


# speedup vs baseline: 10.7038x; 1.1078x over previous
import functools

import jax
import jax.numpy as jnp
from jax import lax
from jax.experimental import pallas as pl
from jax.experimental.pallas import tpu as pltpu, tpu_sc as plsc

N = 10000
E = 320000
D = 128
GSZ = 8000
TAIL = N - GSZ

NC, NS = 2, 16
NW = NC * NS
EPW = E // NW
K = 80
CH = EPW // K
RPT = 624
ZR = 24
REM = N - NS * RPT

MB = 2000
EPB = E // (N // MB)


def _mm_body(notes_ref, w_ref, ei_ref, o_ref, p_ref):
    o_ref[...] = jnp.dot(notes_ref[...], w_ref[...],
                         preferred_element_type=jnp.float32)

    @pl.when(pl.program_id(0) == 0)
    def _():
        p_ref[...] = jnp.bitwise_or(
            jnp.left_shift(ei_ref[0], 16),
            ei_ref[1]).reshape(E // 128, 128)


def _matmul_pack(notes, w, ei):
    return pl.pallas_call(
        _mm_body,
        grid=(N // MB,),
        in_specs=[
            pl.BlockSpec((MB, D), lambda i: (i, 0)),
            pl.BlockSpec((D, D), lambda i: (0, 0)),
            pl.BlockSpec((2, E), lambda i: (0, 0)),
        ],
        out_specs=[
            pl.BlockSpec((MB, D), lambda i: (i, 0)),
            pl.BlockSpec((E // 128, 128), lambda i: (0, 0)),
        ],
        out_shape=[
            jax.ShapeDtypeStruct((N, D), jnp.float32),
            jax.ShapeDtypeStruct((E // 128, 128), jnp.int32),
        ],
    )(notes, w, ei)


def _sc_scatter_fn():
    mesh = plsc.VectorSubcoreMesh(
        core_axis_name="c", subcore_axis_name="s",
        num_cores=NC, num_subcores=NS)

    @functools.partial(
        pl.kernel,
        out_type=jax.ShapeDtypeStruct((NC, N, D), jnp.float32),
        mesh=mesh,
        scratch_types=[
            pltpu.VMEM((EPW,), jnp.int32),
            pltpu.VMEM((K,), jnp.int32),
            pltpu.VMEM((K,), jnp.int32),
            pltpu.VMEM((K,), jnp.int32),
            pltpu.VMEM((K,), jnp.int32),
            pltpu.VMEM((K,), jnp.float32),
            pltpu.VMEM((K,), jnp.float32),
            pltpu.VMEM((K, D), jnp.float32),
            pltpu.VMEM((K, D), jnp.float32),
            pltpu.VMEM((ZR, D), jnp.float32),
            pltpu.VMEM_SHARED((N, D), jnp.float32),
            pltpu.SemaphoreType.DMA,
            pltpu.SemaphoreType.DMA,
            pltpu.SemaphoreType.DMA,
            pltpu.SemaphoreType.DMA,
            pltpu.SemaphoreType.DMA,
        ],
    )
    def sc_scatter(x0_hbm, packed_hbm, ew_hbm, part_hbm,
                   packed_v, src0, src1, dst0, dst1, ew0, ew1,
                   rows0, rows1, zbuf, acc,
                   gsem0, gsem1, ssem0, ssem1, zsem):
        c = lax.axis_index("c")
        s = lax.axis_index("s")
        wid = c * NS + s
        ebase = wid * EPW
        rows = (rows0, rows1)
        srcb = (src0, src1)
        dstb = (dst0, dst1)
        ewb = (ew0, ew1)
        gsems = (gsem0, gsem1)
        ssems = (ssem0, ssem1)

        pltpu.async_copy(packed_hbm.at[pl.ds(ebase, EPW)], packed_v, gsem0)

        @pl.loop(0, ZR)
        def _(r):
            for g in range(D // 16):
                zbuf[r, pl.ds(g * 16, 16)] = jnp.zeros((16,), jnp.float32)

        @pl.loop(0, RPT // ZR)
        def _(j):
            pltpu.async_copy(zbuf, acc.at[pl.ds(s * RPT + j * ZR, ZR)],
                             zsem)

        @pl.when(s == NS - 1)
        def _():
            pltpu.async_copy(zbuf.at[pl.ds(0, REM)],
                            acc.at[pl.ds(NS * RPT, REM)], zsem)

        @pl.loop(0, RPT // ZR)
        def _(j):
            pltpu.make_async_copy(
                zbuf, acc.at[pl.ds(s * RPT + j * ZR, ZR)], zsem).wait()

        @pl.when(s == NS - 1)
        def _():
            pltpu.make_async_copy(
                zbuf.at[pl.ds(0, REM)],
                acc.at[pl.ds(NS * RPT, REM)], zsem).wait()

        pltpu.make_async_copy(
            packed_hbm.at[pl.ds(ebase, EPW)], packed_v, gsem0).wait()

        def unpack(i, b):
            for v in range(K // 16):
                sl = pl.ds(v * 16, 16)
                p = packed_v[pl.ds(i * K + v * 16, 16)]
                srcb[b][sl] = jnp.bitwise_and(p, 0xFFFF)
                dstb[b][sl] = jnp.right_shift(p, 16)

        def issue_gather(i, b):
            pltpu.async_copy(x0_hbm.at[srcb[b]], rows[b], gsems[b])
            pltpu.async_copy(
                ew_hbm.at[pl.ds(ebase + i * K, K)], ewb[b], gsems[b])

        def wait_gather(i, b):
            pltpu.make_async_copy(
                x0_hbm.at[srcb[b]], rows[b], gsems[b]).wait()
            pltpu.make_async_copy(
                ew_hbm.at[pl.ds(ebase + i * K, K)], ewb[b], gsems[b]).wait()

        def issue_scatter(b):
            pltpu.async_copy(rows[b], acc.at[dstb[b]], ssems[b], add=True)

        def wait_scatter(b):
            pltpu.make_async_copy(
                rows[b], acc.at[dstb[b]], ssems[b]).wait()

        def weight_rows(b):
            @pl.loop(0, K // 16, unroll=K // 16)
            def _(eb):
                wchunk = ewb[b][pl.ds(eb * 16, 16)]
                for l in range(16):
                    wv = jnp.full((16,), 0.0, jnp.float32) + wchunk[l]
                    e = eb * 16 + l
                    for g in range(D // 16):
                        sl = pl.ds(g * 16, 16)
                        rows[b][e, sl] = rows[b][e, sl] * wv

        unpack(0, 0)
        issue_gather(0, 0)
        plsc.subcore_barrier()

        wait_gather(0, 0)
        unpack(1, 1)
        issue_gather(1, 1)
        weight_rows(0)
        issue_scatter(0)

        @pl.loop(0, (CH - 1) // 2)
        def _(j):
            for bi in range(2):
                i = 1 + 2 * j + bi
                b = (1 + bi) % 2
                wait_gather(i, b)
                wait_scatter(1 - b)
                i1 = jnp.minimum(i + 1, CH - 1)
                unpack(i1, 1 - b)
                issue_gather(i1, 1 - b)
                weight_rows(b)
                issue_scatter(b)

        wait_scatter(0)
        wait_gather(CH - 1, 1)

        plsc.subcore_barrier()

        pltpu.sync_copy(acc.at[pl.ds(s * RPT, RPT)],
                        part_hbm.at[c, pl.ds(s * RPT, RPT)])

        @pl.when(s == NS - 1)
        def _():
            pltpu.sync_copy(acc.at[pl.ds(NS * RPT, REM)],
                            part_hbm.at[c, pl.ds(NS * RPT, REM)])

    return sc_scatter


_sc_scatter = _sc_scatter_fn()


def _merge_body(parts_ref, notes_ref, w_ref, b_ref, o_ref):
    i = pl.program_id(0)

    @pl.when(i < N // MB)
    def _():
        o_ref[...] = jnp.maximum(
            parts_ref[0] + parts_ref[1] + b_ref[...], 0.0)

    @pl.when(i >= N // MB)
    def _():
        o_ref[...] = jnp.dot(notes_ref[...], w_ref[...],
                             preferred_element_type=jnp.float32)


def _merge(parts, notes, w, b):
    nblk = (N + TAIL) // MB
    return pl.pallas_call(
        _merge_body,
        grid=(nblk,),
        in_specs=[
            pl.BlockSpec((NC, MB, D),
                         lambda i: (0, jnp.minimum(i, N // MB - 1), 0)),
            pl.BlockSpec((MB, D),
                         lambda i: (jnp.where(i >= N // MB, GSZ // MB, 0), 0)),
            pl.BlockSpec((D, D), lambda i: (0, 0)),
            pl.BlockSpec((1, D), lambda i: (0, 0)),
        ],
        out_specs=pl.BlockSpec((MB, D), lambda i: (i, 0)),
        out_shape=jax.ShapeDtypeStruct((N + TAIL, D), jnp.float32),
    )(parts, notes, w, b)


def kernel(notes, edge_index, edge_weight, w, b, garment_size):
    del garment_size
    ei = edge_index.astype(jnp.int32)
    x0, packed = _matmul_pack(notes, w, ei)
    parts = _sc_scatter(x0, packed.reshape(E), edge_weight)
    return _merge(parts, notes, w, b.reshape(1, D))


# speedup vs baseline: 12.1413x; 1.1343x over previous
import functools

import jax
import jax.numpy as jnp
from jax import lax
from jax.experimental import pallas as pl
from jax.experimental.pallas import tpu as pltpu, tpu_sc as plsc

N = 10000
E = 320000
D = 128
GSZ = 8000
TAIL = N - GSZ

NC, NS = 2, 16
NW = NC * NS
EPW = E // NW
K = 128
CHM = EPW // K
TOFF = CHM * K
TK = EPW - TOFF
RPT = 624
ZR = 24
REM = N - NS * RPT

MB = 2000


def _mm_body(notes_ref, w_ref, ei_ref, o_ref, p_ref):
    o_ref[...] = jnp.dot(notes_ref[...], w_ref[...],
                         preferred_element_type=jnp.float32)

    @pl.when(pl.program_id(0) == 0)
    def _():
        p_ref[...] = jnp.bitwise_or(
            jnp.left_shift(ei_ref[0], 16),
            ei_ref[1]).reshape(E // 128, 128)


def _matmul_pack(notes, w, ei):
    return pl.pallas_call(
        _mm_body,
        grid=(N // MB,),
        in_specs=[
            pl.BlockSpec((MB, D), lambda i: (i, 0)),
            pl.BlockSpec((D, D), lambda i: (0, 0)),
            pl.BlockSpec((2, E), lambda i: (0, 0)),
        ],
        out_specs=[
            pl.BlockSpec((MB, D), lambda i: (i, 0)),
            pl.BlockSpec((E // 128, 128), lambda i: (0, 0)),
        ],
        out_shape=[
            jax.ShapeDtypeStruct((N, D), jnp.float32),
            jax.ShapeDtypeStruct((E // 128, 128), jnp.int32),
        ],
    )(notes, w, ei)


def _sc_scatter_fn():
    mesh = plsc.VectorSubcoreMesh(
        core_axis_name="c", subcore_axis_name="s",
        num_cores=NC, num_subcores=NS)

    @functools.partial(
        pl.kernel,
        out_type=jax.ShapeDtypeStruct((NC, N, D), jnp.float32),
        mesh=mesh,
        scratch_types=[
            pltpu.VMEM((EPW,), jnp.int32),
            pltpu.VMEM((K,), jnp.int32),
            pltpu.VMEM((K,), jnp.int32),
            pltpu.VMEM((K,), jnp.int32),
            pltpu.VMEM((K,), jnp.int32),
            pltpu.VMEM((K,), jnp.float32),
            pltpu.VMEM((K,), jnp.float32),
            pltpu.VMEM((TK,), jnp.int32),
            pltpu.VMEM((TK,), jnp.int32),
            pltpu.VMEM((TK,), jnp.float32),
            pltpu.VMEM((K, D), jnp.float32),
            pltpu.VMEM((K, D), jnp.float32),
            pltpu.VMEM((ZR, D), jnp.float32),
            pltpu.VMEM_SHARED((N, D), jnp.float32),
            pltpu.SemaphoreType.DMA,
            pltpu.SemaphoreType.DMA,
            pltpu.SemaphoreType.DMA,
            pltpu.SemaphoreType.DMA,
            pltpu.SemaphoreType.DMA,
        ],
    )
    def sc_scatter(x0_hbm, packed_hbm, ew_hbm, part_hbm,
                   packed_v, src0, src1, dst0, dst1, ew0, ew1,
                   src_t, dst_t, ew_t, rows0, rows1, zbuf, acc,
                   gsem0, gsem1, ssem0, ssem1, zsem):
        c = lax.axis_index("c")
        s = lax.axis_index("s")
        wid = c * NS + s
        ebase = wid * EPW
        rows = (rows0, rows1)
        srcb = (src0, src1)
        dstb = (dst0, dst1)
        ewb = (ew0, ew1)
        gsems = (gsem0, gsem1)
        ssems = (ssem0, ssem1)

        pltpu.async_copy(packed_hbm.at[pl.ds(ebase, EPW)], packed_v, gsem0)

        @pl.loop(0, ZR)
        def _(r):
            for g in range(D // 16):
                zbuf[r, pl.ds(g * 16, 16)] = jnp.zeros((16,), jnp.float32)

        @pl.loop(0, RPT // ZR)
        def _(j):
            pltpu.async_copy(zbuf, acc.at[pl.ds(s * RPT + j * ZR, ZR)],
                             zsem)

        @pl.when(s == NS - 1)
        def _():
            pltpu.async_copy(zbuf.at[pl.ds(0, REM)],
                            acc.at[pl.ds(NS * RPT, REM)], zsem)

        @pl.loop(0, RPT // ZR)
        def _(j):
            pltpu.make_async_copy(
                zbuf, acc.at[pl.ds(s * RPT + j * ZR, ZR)], zsem).wait()

        @pl.when(s == NS - 1)
        def _():
            pltpu.make_async_copy(
                zbuf.at[pl.ds(0, REM)],
                acc.at[pl.ds(NS * RPT, REM)], zsem).wait()

        pltpu.make_async_copy(
            packed_hbm.at[pl.ds(ebase, EPW)], packed_v, gsem0).wait()

        def unpack(i, b):
            for v in range(K // 16):
                sl = pl.ds(v * 16, 16)
                p = packed_v[pl.ds(i * K + v * 16, 16)]
                srcb[b][sl] = jnp.bitwise_and(p, 0xFFFF)
                dstb[b][sl] = jnp.right_shift(p, 16)

        def issue_gather(i, b):
            pltpu.async_copy(x0_hbm.at[srcb[b]], rows[b], gsems[b])
            pltpu.async_copy(
                ew_hbm.at[pl.ds(ebase + i * K, K)], ewb[b], gsems[b])

        def wait_gather(i, b):
            pltpu.make_async_copy(
                x0_hbm.at[srcb[b]], rows[b], gsems[b]).wait()
            pltpu.make_async_copy(
                ew_hbm.at[pl.ds(ebase + i * K, K)], ewb[b], gsems[b]).wait()

        def issue_scatter(b):
            pltpu.async_copy(rows[b], acc.at[dstb[b]], ssems[b], add=True)

        def wait_scatter(b):
            pltpu.make_async_copy(
                rows[b], acc.at[dstb[b]], ssems[b]).wait()

        def weight_rows(b):
            @pl.loop(0, K // 16, unroll=4)
            def _(eb):
                wchunk = ewb[b][pl.ds(eb * 16, 16)]
                for l in range(16):
                    wv = jnp.full((16,), 0.0, jnp.float32) + wchunk[l]
                    e = eb * 16 + l
                    for g in range(D // 16):
                        sl = pl.ds(g * 16, 16)
                        rows[b][e, sl] = rows[b][e, sl] * wv

        unpack(0, 0)
        issue_gather(0, 0)
        plsc.subcore_barrier()

        wait_gather(0, 0)
        unpack(1, 1)
        issue_gather(1, 1)
        weight_rows(0)
        issue_scatter(0)

        @pl.loop(0, (CHM - 2) // 2)
        def _(j):
            for bi in range(2):
                i = 1 + 2 * j + bi
                b = (1 + bi) % 2
                wait_gather(i, b)
                wait_scatter(1 - b)
                unpack(i + 1, 1 - b)
                issue_gather(i + 1, 1 - b)
                weight_rows(b)
                issue_scatter(b)

        wait_gather(CHM - 1, 1)
        wait_scatter(0)
        weight_rows(1)
        issue_scatter(1)

        p_t = packed_v[pl.ds(TOFF, TK)]
        src_t[...] = jnp.bitwise_and(p_t, 0xFFFF)
        dst_t[...] = jnp.right_shift(p_t, 16)
        pltpu.sync_copy(ew_hbm.at[pl.ds(ebase + TOFF, TK)], ew_t)
        pltpu.async_copy(x0_hbm.at[src_t], rows0.at[pl.ds(0, TK)],
                         gsem0).wait()
        wtail = ew_t[...]
        for l in range(TK):
            wv = jnp.full((16,), 0.0, jnp.float32) + wtail[l]
            for g in range(D // 16):
                sl = pl.ds(g * 16, 16)
                rows0[l, sl] = rows0[l, sl] * wv
        pltpu.sync_copy(rows0.at[pl.ds(0, TK)], acc.at[dst_t], add=True)

        wait_scatter(1)

        plsc.subcore_barrier()

        pltpu.sync_copy(acc.at[pl.ds(s * RPT, RPT)],
                        part_hbm.at[c, pl.ds(s * RPT, RPT)])

        @pl.when(s == NS - 1)
        def _():
            pltpu.sync_copy(acc.at[pl.ds(NS * RPT, REM)],
                            part_hbm.at[c, pl.ds(NS * RPT, REM)])

    return sc_scatter


_sc_scatter = _sc_scatter_fn()


def _merge_body(parts_ref, notes_ref, w_ref, b_ref, o_ref):
    i = pl.program_id(0)

    @pl.when(i < N // MB)
    def _():
        o_ref[...] = jnp.maximum(
            parts_ref[0] + parts_ref[1] + b_ref[...], 0.0)

    @pl.when(i >= N // MB)
    def _():
        o_ref[...] = jnp.dot(notes_ref[...], w_ref[...],
                             preferred_element_type=jnp.float32)


def _merge(parts, notes, w, b):
    nblk = (N + TAIL) // MB
    return pl.pallas_call(
        _merge_body,
        grid=(nblk,),
        in_specs=[
            pl.BlockSpec((NC, MB, D),
                         lambda i: (0, jnp.minimum(i, N // MB - 1), 0)),
            pl.BlockSpec((MB, D),
                         lambda i: (jnp.where(i >= N // MB, GSZ // MB, 0), 0)),
            pl.BlockSpec((D, D), lambda i: (0, 0)),
            pl.BlockSpec((1, D), lambda i: (0, 0)),
        ],
        out_specs=pl.BlockSpec((MB, D), lambda i: (i, 0)),
        out_shape=jax.ShapeDtypeStruct((N + TAIL, D), jnp.float32),
    )(parts, notes, w, b)


def kernel(notes, edge_index, edge_weight, w, b, garment_size):
    del garment_size
    ei = edge_index.astype(jnp.int32)
    x0, packed = _matmul_pack(notes, w, ei)
    parts = _sc_scatter(x0, packed.reshape(E), edge_weight)
    return _merge(parts, notes, w, b.reshape(1, D))
